# baseline (speedup 1.0000x reference)
import jax
import jax.numpy as jnp
from jax import lax
from jax.experimental import pallas as pl
from jax.experimental.pallas import tpu as pltpu
from jax.experimental.pallas import tpu_sc as plsc

B = 16384
D = 32
L = 16
NC = 2
NS = 16
NW = NC * NS
NUSERS = 1000000
UNITW = 768
SWEPT = 999936
NUNITS = SWEPT // UNITW
UPT = 41
TAILN = NUSERS - SWEPT
RCAP = 48
NRUNS = 42
RUNTOT = 2048
RW = 49
UCAP = 768
NGRP = UCAP // L
OUTPAD = 128

_i32 = jnp.int32
_f32 = jnp.float32


def _popcnt(mask):
    return plsc.all_reduce_population_count(mask)[0]


def _p0_body(uid_hbm, uruns_hbm, eruns_hbm,
             uid_v, ulist_v, elist_v, urun_v, erun_v, sem):
    c = lax.axis_index("c")
    s = lax.axis_index("s")
    T = s * NC + c
    lane = lax.iota(_i32, L)

    pltpu.async_copy(uid_hbm.at[pl.ds(0, B)], uid_v, sem).wait()

    u0 = T * UPT
    lo = u0 * UNITW
    nu = jnp.where(T == NW - 1, NUNITS - (NW - 1) * UPT, UPT)
    hi = lo + nu * UNITW

    neg16 = jnp.full((L,), -1, _i32)
    def init_body(k, carry):
        urun_v[pl.ds(k * L, L)] = neg16
        erun_v[pl.ds(k * L, L)] = neg16
        return carry
    lax.fori_loop(0, RUNTOT // L, init_body, 0)

    tail_tile = T == NW - 1
    def scan_body(i, off):
        u16 = uid_v[pl.ds(i * L, L)]
        e16 = i * L + lane
        m = (u16 >= lo) & (u16 < hi)
        m = m | (tail_tile & (u16 >= SWEPT))
        plsc.store_compressed(ulist_v.at[pl.ds(off, L)], u16, mask=m)
        plsc.store_compressed(elist_v.at[pl.ds(off, L)], e16, mask=m)
        return off + _popcnt(m)
    count = lax.fori_loop(0, B // L, scan_body, 0)

    def unit_body(un, carry):
        base = jnp.where(un == NRUNS - 1, SWEPT, (u0 + un) * UNITW)
        top = jnp.where(un == NRUNS - 1, NUSERS, base + UNITW)
        def rescan(k, uoff):
            lu = ulist_v[pl.ds(k * L, L)]
            le = elist_v[pl.ds(k * L, L)]
            m = ((k * L + lane) < count) & (lu >= base) & (lu < top)
            uoff_c = jnp.clip(uoff, 0, RCAP - L)
            plsc.store_compressed(
                urun_v.at[pl.ds(un * RCAP + uoff_c, L)], lu, mask=m)
            plsc.store_compressed(
                erun_v.at[pl.ds(un * RCAP + uoff_c, L)], le, mask=m)
            return uoff + _popcnt(m)
        lax.fori_loop(0, (count + L - 1) // L, rescan, 0)
        return carry
    lax.fori_loop(0, NRUNS, unit_body, 0)

    pltpu.sync_copy(urun_v, uruns_hbm.at[pl.ds(T * RUNTOT, RUNTOT)])
    pltpu.sync_copy(erun_v, eruns_hbm.at[pl.ds(T * RUNTOT, RUNTOT)])


def _p1_body(mid_hbm, uembT_hbm, utail_hbm, uruns_hbm, eruns_hbm, rec_hbm,
             mid_v, utail_v, urun_v, erun_v, stage_v, rec_v, sem):
    c = lax.axis_index("c")
    s = lax.axis_index("s")
    T = s * NC + c
    lane = lax.iota(_i32, L)

    cps = [
        pltpu.async_copy(mid_hbm.at[pl.ds(0, B)], mid_v, sem),
        pltpu.async_copy(utail_hbm.at[pl.ds(0, TAILN * D)], utail_v, sem),
        pltpu.async_copy(uruns_hbm.at[pl.ds(T * RUNTOT, RUNTOT)], urun_v, sem),
        pltpu.async_copy(eruns_hbm.at[pl.ds(T * RUNTOT, RUNTOT)], erun_v, sem),
    ]
    for cp in cps:
        cp.wait()

    u0 = T * UPT
    nu = jnp.where(T == NW - 1, NUNITS - (NW - 1) * UPT, UPT)

    zero16 = jnp.zeros((L,), _f32)
    neg16 = plsc.bitcast(jnp.full((L,), -1, _i32), _f32)
    def init_body(k, carry):
        ibase = k * RW * L
        rec_v[pl.ds(ibase + 32 * L, L)] = zero16
        rec_v[pl.ds(ibase + 33 * L, L)] = neg16
        rec_v[pl.ds(ibase + 34 * L, L)] = zero16
        return carry
    lax.fori_loop(0, NGRP, init_body, 0)

    def fire(un, buf):
        base = jnp.minimum((u0 + un) * UNITW, SWEPT - UNITW)
        for t in range(4):
            pltpu.async_copy(
                uembT_hbm.at[pl.ds(8 * t, 8), pl.ds(base, UNITW)],
                stage_v.at[buf].at[pl.ds(8 * t, 8)], sem)

    def drain_unit():
        for t in range(4):
            pltpu.make_async_copy(
                uembT_hbm.at[pl.ds(0, 8), pl.ds(0, UNITW)],
                stage_v.at[0].at[pl.ds(8 * t, 8)], sem).wait()

    def emit_records(lu, le, valid, vals_fn, off):
        li = jnp.clip(off + plsc.cumsum(valid.astype(_i32)) - 1, 0, UCAP - 1)
        rbase = (li // L) * (RW * L) + (li % L)
        for w in range(D):
            plsc.store_scatter(
                rec_v, [rbase + w * L], vals_fn(w), mask=valid)
        plsc.store_scatter(
            rec_v, [rbase + 32 * L], plsc.bitcast(lu, _f32), mask=valid)
        plsc.store_scatter(
            rec_v, [rbase + 33 * L], plsc.bitcast(le, _f32), mask=valid)
        lm = plsc.load_gather(mid_v, [jnp.clip(le, 0, B - 1)])
        plsc.store_scatter(
            rec_v, [rbase + 34 * L], plsc.bitcast(lm, _f32), mask=valid)
        return off + _popcnt(valid)

    fire(0, 0)

    def unit_body(un, off):
        buf = un % 2
        fire(jnp.minimum(un + 1, UPT - 1), (un + 1) % 2)
        drain_unit()
        base = (u0 + un) * UNITW
        bcast = jnp.full((L,), buf, _i32)
        for k in range(RCAP // L):
            lu = urun_v[pl.ds(un * RCAP + k * L, L)]
            le = erun_v[pl.ds(un * RCAP + k * L, L)]
            valid = lu >= 0
            nv = _popcnt(valid)
            j = jnp.clip(lu - base, 0, UNITW - 1)

            @pl.when(nv > 0)
            def _emit(lu=lu, le=le, valid=valid, j=j, off=off):
                emit_records(
                    lu, le, valid,
                    lambda w: plsc.load_gather(
                        stage_v,
                        [bcast, jnp.full((L,), w, _i32), j]),
                    off)
            off = off + nv
        return off

    off = lax.fori_loop(0, nu, unit_body, 0)
    drain_unit()

    for k in range(RCAP // L):
        lu = urun_v[pl.ds((NRUNS - 1) * RCAP + k * L, L)]
        le = erun_v[pl.ds((NRUNS - 1) * RCAP + k * L, L)]
        valid = lu >= 0
        nv = _popcnt(valid)
        jt = jnp.clip(lu - SWEPT, 0, TAILN - 1) * D

        @pl.when(nv > 0)
        def _emit_tail(lu=lu, le=le, valid=valid, jt=jt, off=off):
            emit_records(
                lu, le, valid,
                lambda w: plsc.load_gather(utail_v, [jt + w]),
                off)
        off = off + nv

    pltpu.sync_copy(rec_v, rec_hbm.at[pl.ds(T * UCAP * RW, UCAP * RW)])


def _p2_body(rec_hbm, memb_hbm, ubias_hbm, mbias_hbm, res_hbm, e_hbm,
             rec_v, mrows_v, widx_v, uidx_v, midx_v, e2_v, ubias_v, mbias_v,
             res_v, sem):
    c = lax.axis_index("c")
    s = lax.axis_index("s")
    T = s * NC + c
    lane = lax.iota(_i32, L)

    pltpu.sync_copy(rec_hbm.at[pl.ds(T * UCAP * RW, UCAP * RW)], rec_v)

    def unpack(g, carry):
        gbase = g * RW * L
        uid = plsc.bitcast(rec_v[pl.ds(gbase + 32 * L, L)], _i32)
        e = plsc.bitcast(rec_v[pl.ds(gbase + 33 * L, L)], _i32)
        mid = plsc.bitcast(rec_v[pl.ds(gbase + 34 * L, L)], _i32)
        uidx_v[pl.ds(g * L, L)] = uid
        midx_v[pl.ds(g * L, L)] = mid
        e2_v[pl.ds(g * L, L)] = jnp.where(e < 0, B, e)
        base = mid * D
        for d in range(D):
            widx_v[pl.ds(g * D * L + d * L, L)] = base + d
        return carry
    lax.fori_loop(0, NGRP, unpack, 0)

    pend = []
    for j in range(UCAP * D // 128):
        sl = pl.ds(j * 128, 128)
        pend.append(pltpu.async_copy(
            memb_hbm.at[widx_v.at[sl]], mrows_v.at[sl], sem))
    for j in range(UCAP // 128):
        sl = pl.ds(j * 128, 128)
        pend.append(pltpu.async_copy(
            ubias_hbm.at[uidx_v.at[sl]], ubias_v.at[sl], sem))
        pend.append(pltpu.async_copy(
            mbias_hbm.at[midx_v.at[sl]], mbias_v.at[sl], sem))
    for cp in pend:
        cp.wait()

    def group_body(g, carry):
        li = g * L + lane
        gbase = g * RW * L
        acc = ubias_v[pl.ds(g * L, L)] + mbias_v[pl.ds(g * L, L)]
        for d in range(D):
            uu = rec_v[pl.ds(gbase + d * L, L)]
            mm = mrows_v[pl.ds(g * D * L + d * L, L)]
            acc = acc + uu * mm
        res_v[pl.ds(g * L, L)] = acc
        return carry
    lax.fori_loop(0, NGRP, group_body, 0)

    cp0 = pltpu.async_copy(res_v, res_hbm.at[pl.ds(T * UCAP, UCAP)], sem)
    cp1 = pltpu.async_copy(e2_v, e_hbm.at[pl.ds(T * UCAP, UCAP)], sem)
    cp0.wait()
    cp1.wait()


def _p3_body(res_hbm, e_hbm, out_hbm, resa_v, ea_v, out_v, sem):
    c = lax.axis_index("c")
    s = lax.axis_index("s")
    T = s * NC + c
    lane = lax.iota(_i32, L)
    opt = B // NW

    cp0 = pltpu.async_copy(res_hbm.at[pl.ds(0, NW * UCAP)], resa_v, sem)
    cp1 = pltpu.async_copy(e_hbm.at[pl.ds(0, NW * UCAP)], ea_v, sem)
    cp0.wait()
    cp1.wait()

    tlo = T * opt
    def pick(i, carry):
        e = ea_v[pl.ds(i * L, L)]
        r = resa_v[pl.ds(i * L, L)]
        m = (e >= tlo) & (e < tlo + opt)
        plsc.store_scatter(
            out_v, [jnp.clip(e - tlo, 0, opt - 1)], r, mask=m)
        return carry
    lax.fori_loop(0, NW * UCAP // L, pick, 0)

    pltpu.sync_copy(out_v, out_hbm.at[pl.ds(tlo, opt)])


@jax.jit
def kernel(user_ids, movie_ids, user_emb, movie_emb, user_bias, movie_bias):
    mesh = plsc.VectorSubcoreMesh(core_axis_name="c", subcore_axis_name="s")
    p0 = pl.kernel(
        _p0_body,
        mesh=mesh,
        compiler_params=pltpu.CompilerParams(
            needs_layout_passes=False, use_tc_tiling_on_sc=False),
        out_type=[
            jax.ShapeDtypeStruct((NW * RUNTOT,), _i32),
            jax.ShapeDtypeStruct((NW * RUNTOT,), _i32),
        ],
        scratch_types=[
            pltpu.VMEM((B,), _i32),
            pltpu.VMEM((UCAP,), _i32),
            pltpu.VMEM((UCAP,), _i32),
            pltpu.VMEM((RUNTOT,), _i32),
            pltpu.VMEM((RUNTOT,), _i32),
            pltpu.SemaphoreType.DMA,
        ],
    )
    p1 = pl.kernel(
        _p1_body,
        mesh=mesh,
        compiler_params=pltpu.CompilerParams(
            needs_layout_passes=False, use_tc_tiling_on_sc=True),
        out_type=jax.ShapeDtypeStruct((NW * UCAP * RW,), _f32),
        scratch_types=[
            pltpu.VMEM((B,), _i32),
            pltpu.VMEM((TAILN * D,), _f32),
            pltpu.VMEM((RUNTOT,), _i32),
            pltpu.VMEM((RUNTOT,), _i32),
            pltpu.VMEM((2, 32, UNITW), _f32),
            pltpu.VMEM((UCAP * RW,), _f32),
            pltpu.SemaphoreType.DMA,
        ],
    )
    p2 = pl.kernel(
        _p2_body,
        mesh=mesh,
        compiler_params=pltpu.CompilerParams(
            needs_layout_passes=False, use_tc_tiling_on_sc=False),
        out_type=[
            jax.ShapeDtypeStruct((NW * UCAP,), _f32),
            jax.ShapeDtypeStruct((NW * UCAP,), _i32),
        ],
        scratch_types=[
            pltpu.VMEM((UCAP * RW,), _f32),
            pltpu.VMEM((UCAP * D,), _f32),
            pltpu.VMEM((UCAP * D,), _i32),
            pltpu.VMEM((UCAP,), _i32),
            pltpu.VMEM((UCAP,), _i32),
            pltpu.VMEM((UCAP,), _i32),
            pltpu.VMEM((UCAP,), _f32),
            pltpu.VMEM((UCAP,), _f32),
            pltpu.VMEM((UCAP,), _f32),
            pltpu.SemaphoreType.DMA,
        ],
    )
    p3 = pl.kernel(
        _p3_body,
        mesh=mesh,
        compiler_params=pltpu.CompilerParams(
            needs_layout_passes=False, use_tc_tiling_on_sc=False),
        out_type=jax.ShapeDtypeStruct((B,), _f32),
        scratch_types=[
            pltpu.VMEM((NW * UCAP,), _f32),
            pltpu.VMEM((NW * UCAP,), _i32),
            pltpu.VMEM((B // NW,), _f32),
            pltpu.SemaphoreType.DMA,
        ],
    )
    uids = user_ids.astype(_i32)
    utail = user_emb[SWEPT:, :].reshape(-1)
    uruns, eruns = p0(uids)
    recs = p1(movie_ids.astype(_i32), user_emb.T, utail, uruns, eruns)
    res_all, e_all = p2(recs, movie_emb.reshape(-1), user_bias.reshape(-1),
                        movie_bias.reshape(-1))
    return p3(res_all, e_all)


# speedup vs baseline: 5.4960x; 5.4960x over previous
import jax
import jax.numpy as jnp
from jax import lax
from jax.experimental import pallas as pl
from jax.experimental.pallas import tpu as pltpu
from jax.experimental.pallas import tpu_sc as plsc

B = 16384
D = 32
L = 16
NC = 2
NS = 16
NW = NC * NS
NUSERS = 1000000
UNITW = 768
SWEPT = 999936
NUNITS = SWEPT // UNITW
UPT = 41
TAILN = NUSERS - SWEPT
RCAP = 48
NRUNS = 42
RUNTOT = 2048
RW = 49
UCAP = 768
NGRP = UCAP // L
OUTPAD = 128

_i32 = jnp.int32
_f32 = jnp.float32


def _popcnt(mask):
    return plsc.all_reduce_population_count(mask)[0]


def _p0_body(uid_hbm, uruns_hbm, eruns_hbm,
             uid_v, ulist_v, elist_v, urun_v, erun_v, sem):
    c = lax.axis_index("c")
    s = lax.axis_index("s")
    T = s * NC + c
    lane = lax.iota(_i32, L)

    pltpu.async_copy(uid_hbm.at[pl.ds(0, B)], uid_v, sem).wait()

    u0 = T * UPT
    lo = u0 * UNITW
    nu = jnp.where(T == NW - 1, NUNITS - (NW - 1) * UPT, UPT)
    hi = lo + nu * UNITW

    neg16 = jnp.full((L,), -1, _i32)
    def init_body(k, carry):
        urun_v[pl.ds(k * L, L)] = neg16
        erun_v[pl.ds(k * L, L)] = neg16
        return carry
    lax.fori_loop(0, RUNTOT // L, init_body, 0)

    tail_tile = T == NW - 1
    def scan_body(i, off):
        u16 = uid_v[pl.ds(i * L, L)]
        e16 = i * L + lane
        m = (u16 >= lo) & (u16 < hi)
        m = m | (tail_tile & (u16 >= SWEPT))
        plsc.store_compressed(ulist_v.at[pl.ds(off, L)], u16, mask=m)
        plsc.store_compressed(elist_v.at[pl.ds(off, L)], e16, mask=m)
        return off + _popcnt(m)
    count = lax.fori_loop(0, B // L, scan_body, 0)

    def unit_body(un, carry):
        base = jnp.where(un == NRUNS - 1, SWEPT, (u0 + un) * UNITW)
        top = jnp.where(un == NRUNS - 1, NUSERS, base + UNITW)
        def rescan(k, uoff):
            lu = ulist_v[pl.ds(k * L, L)]
            le = elist_v[pl.ds(k * L, L)]
            m = ((k * L + lane) < count) & (lu >= base) & (lu < top)
            uoff_c = jnp.clip(uoff, 0, RCAP - L)
            plsc.store_compressed(
                urun_v.at[pl.ds(un * RCAP + uoff_c, L)], lu, mask=m)
            plsc.store_compressed(
                erun_v.at[pl.ds(un * RCAP + uoff_c, L)], le, mask=m)
            return uoff + _popcnt(m)
        lax.fori_loop(0, (count + L - 1) // L, rescan, 0)
        return carry
    lax.fori_loop(0, NRUNS, unit_body, 0)

    pltpu.sync_copy(urun_v, uruns_hbm.at[pl.ds(T * RUNTOT, RUNTOT)])
    pltpu.sync_copy(erun_v, eruns_hbm.at[pl.ds(T * RUNTOT, RUNTOT)])


def _p1_body(mid_hbm, uembT_hbm, utail_hbm, uruns_hbm, eruns_hbm, rec_hbm,
             mid_v, utail_v, urun_v, erun_v, stage_v, rec_v, sem):
    c = lax.axis_index("c")
    s = lax.axis_index("s")
    T = s * NC + c
    lane = lax.iota(_i32, L)

    cps = [
        pltpu.async_copy(mid_hbm.at[pl.ds(0, B)], mid_v, sem),
        pltpu.async_copy(utail_hbm.at[pl.ds(0, TAILN * D)], utail_v, sem),
        pltpu.async_copy(uruns_hbm.at[pl.ds(T * RUNTOT, RUNTOT)], urun_v, sem),
        pltpu.async_copy(eruns_hbm.at[pl.ds(T * RUNTOT, RUNTOT)], erun_v, sem),
    ]
    for cp in cps:
        cp.wait()

    u0 = T * UPT
    nu = jnp.where(T == NW - 1, NUNITS - (NW - 1) * UPT, UPT)

    zero16 = jnp.zeros((L,), _f32)
    neg16 = plsc.bitcast(jnp.full((L,), -1, _i32), _f32)
    def init_body(k, carry):
        ibase = k * RW * L
        rec_v[pl.ds(ibase + 32 * L, L)] = zero16
        rec_v[pl.ds(ibase + 33 * L, L)] = neg16
        rec_v[pl.ds(ibase + 34 * L, L)] = zero16
        return carry
    lax.fori_loop(0, NGRP, init_body, 0)

    def fire(un, buf):
        base = jnp.minimum((u0 + un) * UNITW, SWEPT - UNITW)
        for t in range(4):
            pltpu.async_copy(
                uembT_hbm.at[pl.ds(8 * t, 8), pl.ds(base, UNITW)],
                stage_v.at[buf].at[pl.ds(8 * t, 8)], sem)

    def drain_unit():
        for t in range(4):
            pltpu.make_async_copy(
                uembT_hbm.at[pl.ds(0, 8), pl.ds(0, UNITW)],
                stage_v.at[0].at[pl.ds(8 * t, 8)], sem).wait()

    def emit_records(lu, le, valid, vals_fn, off):
        li = jnp.clip(off + plsc.cumsum(valid.astype(_i32)) - 1, 0, UCAP - 1)
        rbase = (li // L) * (RW * L) + (li % L)
        for w in range(D):
            plsc.store_scatter(
                rec_v, [rbase + w * L], vals_fn(w), mask=valid)
        plsc.store_scatter(
            rec_v, [rbase + 32 * L], plsc.bitcast(lu, _f32), mask=valid)
        plsc.store_scatter(
            rec_v, [rbase + 33 * L], plsc.bitcast(le, _f32), mask=valid)
        lm = plsc.load_gather(mid_v, [jnp.clip(le, 0, B - 1)])
        plsc.store_scatter(
            rec_v, [rbase + 34 * L], plsc.bitcast(lm, _f32), mask=valid)
        return off + _popcnt(valid)

    fire(0, 0)

    def unit_body(un, off):
        buf = un % 2
        fire(jnp.minimum(un + 1, UPT - 1), (un + 1) % 2)
        drain_unit()
        base = (u0 + un) * UNITW
        bcast = jnp.full((L,), buf, _i32)
        for k in range(RCAP // L):
            lu = urun_v[pl.ds(un * RCAP + k * L, L)]
            le = erun_v[pl.ds(un * RCAP + k * L, L)]
            valid = lu >= 0
            nv = _popcnt(valid)
            j = jnp.clip(lu - base, 0, UNITW - 1)

            @pl.when(nv > 0)
            def _emit(lu=lu, le=le, valid=valid, j=j, off=off):
                emit_records(
                    lu, le, valid,
                    lambda w: plsc.load_gather(
                        stage_v,
                        [bcast, jnp.full((L,), w, _i32), j]),
                    off)
            off = off + nv
        return off

    off = lax.fori_loop(0, nu, unit_body, 0)
    drain_unit()

    for k in range(RCAP // L):
        lu = urun_v[pl.ds((NRUNS - 1) * RCAP + k * L, L)]
        le = erun_v[pl.ds((NRUNS - 1) * RCAP + k * L, L)]
        valid = lu >= 0
        nv = _popcnt(valid)
        jt = jnp.clip(lu - SWEPT, 0, TAILN - 1) * D

        @pl.when(nv > 0)
        def _emit_tail(lu=lu, le=le, valid=valid, jt=jt, off=off):
            emit_records(
                lu, le, valid,
                lambda w: plsc.load_gather(utail_v, [jt + w]),
                off)
        off = off + nv

    pltpu.sync_copy(rec_v, rec_hbm.at[pl.ds(T * UCAP * RW, UCAP * RW)])


def _p2_body(rec_hbm, memb_hbm, ubias_hbm, mbias_hbm, res_hbm, e_hbm,
             rec_v, mrows_v, uidx_v, midx_v, e2_v, ubias_v, mbias_v,
             res_v, sem):
    c = lax.axis_index("c")
    s = lax.axis_index("s")
    T = s * NC + c
    lane = lax.iota(_i32, L)

    pltpu.sync_copy(rec_hbm.at[pl.ds(T * UCAP * RW, UCAP * RW)], rec_v)

    def unpack(g, carry):
        gbase = g * RW * L
        uid = plsc.bitcast(rec_v[pl.ds(gbase + 32 * L, L)], _i32)
        e = plsc.bitcast(rec_v[pl.ds(gbase + 33 * L, L)], _i32)
        mid = plsc.bitcast(rec_v[pl.ds(gbase + 34 * L, L)], _i32)
        uidx_v[pl.ds(g * L, L)] = uid
        midx_v[pl.ds(g * L, L)] = mid
        e2_v[pl.ds(g * L, L)] = jnp.where(e < 0, B, e)
        return carry
    lax.fori_loop(0, NGRP, unpack, 0)

    pend = []
    for j in range(UCAP // 128):
        sl = pl.ds(j * 128, 128)
        pend.append(pltpu.async_copy(
            memb_hbm.at[midx_v.at[sl]], mrows_v.at[sl], sem))
        pend.append(pltpu.async_copy(
            ubias_hbm.at[uidx_v.at[sl]], ubias_v.at[sl], sem))
        pend.append(pltpu.async_copy(
            mbias_hbm.at[midx_v.at[sl]], mbias_v.at[sl], sem))
    for cp in pend:
        cp.wait()

    def group_body(g, carry):
        li = g * L + lane
        gbase = g * RW * L
        acc = ubias_v[pl.ds(g * L, L)] + mbias_v[pl.ds(g * L, L)]
        for d in range(D):
            uu = rec_v[pl.ds(gbase + d * L, L)]
            mm = plsc.load_gather(mrows_v, [li, jnp.full((L,), d, _i32)])
            acc = acc + uu * mm
        res_v[pl.ds(g * L, L)] = acc
        return carry
    lax.fori_loop(0, NGRP, group_body, 0)

    cp0 = pltpu.async_copy(res_v, res_hbm.at[pl.ds(T * UCAP, UCAP)], sem)
    cp1 = pltpu.async_copy(e2_v, e_hbm.at[pl.ds(T * UCAP, UCAP)], sem)
    cp0.wait()
    cp1.wait()


def _p3_body(res_hbm, e_hbm, out_hbm, resa_v, ea_v, out_v, sem):
    c = lax.axis_index("c")
    s = lax.axis_index("s")
    T = s * NC + c
    lane = lax.iota(_i32, L)
    opt = B // NW

    cp0 = pltpu.async_copy(res_hbm.at[pl.ds(0, NW * UCAP)], resa_v, sem)
    cp1 = pltpu.async_copy(e_hbm.at[pl.ds(0, NW * UCAP)], ea_v, sem)
    cp0.wait()
    cp1.wait()

    tlo = T * opt
    def pick(i, carry):
        e = ea_v[pl.ds(i * L, L)]
        r = resa_v[pl.ds(i * L, L)]
        m = (e >= tlo) & (e < tlo + opt)
        plsc.store_scatter(
            out_v, [jnp.clip(e - tlo, 0, opt - 1)], r, mask=m)
        return carry
    lax.fori_loop(0, NW * UCAP // L, pick, 0)

    pltpu.sync_copy(out_v, out_hbm.at[pl.ds(tlo, opt)])


@jax.jit
def kernel(user_ids, movie_ids, user_emb, movie_emb, user_bias, movie_bias):
    mesh = plsc.VectorSubcoreMesh(core_axis_name="c", subcore_axis_name="s")
    p0 = pl.kernel(
        _p0_body,
        mesh=mesh,
        compiler_params=pltpu.CompilerParams(
            needs_layout_passes=False, use_tc_tiling_on_sc=False),
        out_type=[
            jax.ShapeDtypeStruct((NW * RUNTOT,), _i32),
            jax.ShapeDtypeStruct((NW * RUNTOT,), _i32),
        ],
        scratch_types=[
            pltpu.VMEM((B,), _i32),
            pltpu.VMEM((UCAP,), _i32),
            pltpu.VMEM((UCAP,), _i32),
            pltpu.VMEM((RUNTOT,), _i32),
            pltpu.VMEM((RUNTOT,), _i32),
            pltpu.SemaphoreType.DMA,
        ],
    )
    p1 = pl.kernel(
        _p1_body,
        mesh=mesh,
        compiler_params=pltpu.CompilerParams(
            needs_layout_passes=False, use_tc_tiling_on_sc=True),
        out_type=jax.ShapeDtypeStruct((NW * UCAP * RW,), _f32),
        scratch_types=[
            pltpu.VMEM((B,), _i32),
            pltpu.VMEM((TAILN * D,), _f32),
            pltpu.VMEM((RUNTOT,), _i32),
            pltpu.VMEM((RUNTOT,), _i32),
            pltpu.VMEM((2, 32, UNITW), _f32),
            pltpu.VMEM((UCAP * RW,), _f32),
            pltpu.SemaphoreType.DMA,
        ],
    )
    p2 = pl.kernel(
        _p2_body,
        mesh=mesh,
        compiler_params=pltpu.CompilerParams(
            needs_layout_passes=False, use_tc_tiling_on_sc=False),
        out_type=[
            jax.ShapeDtypeStruct((NW * UCAP,), _f32),
            jax.ShapeDtypeStruct((NW * UCAP,), _i32),
        ],
        scratch_types=[
            pltpu.VMEM((UCAP * RW,), _f32),
            pltpu.VMEM((UCAP, D), _f32),
            pltpu.VMEM((UCAP,), _i32),
            pltpu.VMEM((UCAP,), _i32),
            pltpu.VMEM((UCAP,), _i32),
            pltpu.VMEM((UCAP,), _f32),
            pltpu.VMEM((UCAP,), _f32),
            pltpu.VMEM((UCAP,), _f32),
            pltpu.SemaphoreType.DMA,
        ],
    )
    p3 = pl.kernel(
        _p3_body,
        mesh=mesh,
        compiler_params=pltpu.CompilerParams(
            needs_layout_passes=False, use_tc_tiling_on_sc=False),
        out_type=jax.ShapeDtypeStruct((B,), _f32),
        scratch_types=[
            pltpu.VMEM((NW * UCAP,), _f32),
            pltpu.VMEM((NW * UCAP,), _i32),
            pltpu.VMEM((B // NW,), _f32),
            pltpu.SemaphoreType.DMA,
        ],
    )
    uids = user_ids.astype(_i32)
    utail = user_emb[SWEPT:, :].reshape(-1)
    uruns, eruns = p0(uids)
    recs = p1(movie_ids.astype(_i32), user_emb.T, utail, uruns, eruns)
    res_all, e_all = p2(recs, movie_emb, user_bias.reshape(-1),
                        movie_bias.reshape(-1))
    return p3(res_all, e_all)
